# baseline (speedup 1.0000x reference)
import functools

import jax
import jax.numpy as jnp
from jax import lax
from jax.experimental import pallas as pl
from jax.experimental.pallas import tpu as pltpu
from jax.experimental.pallas import tpu_sc as plsc

_N = 10000
_E = 320000
_D_IN = 128
_HID = 64
_D_OUT = 128

_NC = 2
_NS = 16
_NW = _NC * _NS
_SUB = 128
_SUBS_PER_W = 80
_EPAD = _NW * _SUBS_PER_W * _SUB
_NBUF = 2
_GRP = 2 * _NBUF
_NITER = _SUBS_PER_W // _GRP
_NPAD = 10240
_ROWS_PER_TILE = _NPAD // _NS
_CHUNK = 32


def _make_seg_kernel(with_deg: bool):
  mesh = plsc.VectorSubcoreMesh(core_axis_name="c", subcore_axis_name="s")

  out_type = [jax.ShapeDtypeStruct((_NC * _NPAD, _HID), jnp.float32)]
  scratch = [
      pltpu.VMEM_SHARED((_NPAD, _HID), jnp.float32),
      pltpu.VMEM_SHARED((_NPAD, _HID), jnp.float32),
      pltpu.VMEM((_GRP * _SUB,), jnp.int32),
      pltpu.VMEM((_GRP, _SUB), jnp.int32),
      [pltpu.VMEM((_NBUF * _SUB, _HID), jnp.float32) for _ in range(2)],
      [pltpu.SemaphoreType.DMA, pltpu.SemaphoreType.DMA],
  ]
  if with_deg:
    out_type.append(jax.ShapeDtypeStruct((_NC * _NPAD, 16), jnp.float32))
    scratch += [
        pltpu.VMEM_SHARED((_NPAD, 16), jnp.float32),
        pltpu.VMEM((_SUB, 16), jnp.float32),
    ]
  else:
    out_type.append(jax.ShapeDtypeStruct((_NPAD, _HID), jnp.float32))
    scratch += [
        pltpu.VMEM((_CHUNK, _HID), jnp.float32),
        pltpu.VMEM((_CHUNK, _HID), jnp.float32),
        pltpu.VMEM((_CHUNK, _HID), jnp.float32),
        pltpu.VMEM((_CHUNK, _HID), jnp.float32),
        pltpu.VMEM((_CHUNK, 16), jnp.float32),
        pltpu.VMEM((_CHUNK, 16), jnp.float32),
        pltpu.VMEM((_HID,), jnp.float32),
    ]

  def body(*refs):
    if with_deg:
      (y_hbm, src_hbm, dst_hbm, z64_hbm, z8_hbm, ones_hbm,
       out_hbm, deg_hbm, acc, tbl, srcb, dstb, rows, sems, dacc, onesb) = refs
    else:
      (s1_hbm, degp_hbm, xr_hbm, b_hbm, src_hbm, dst_hbm, z64_hbm,
       out_hbm, h_hbm, acc, tbl, srcb, dstb, rows, sems,
       p0c, p1c, xrc, hc, d0c, d1c, bbuf) = refs
    c = lax.axis_index("c")
    s = lax.axis_index("s")
    wid = s * _NC + c
    r0 = s * _ROWS_PER_TILE

    pltpu.sync_copy(z64_hbm.at[pl.ds(r0, _ROWS_PER_TILE)],
                    acc.at[pl.ds(r0, _ROWS_PER_TILE)])
    if with_deg:
      pltpu.sync_copy(y_hbm.at[pl.ds(r0, _ROWS_PER_TILE)],
                      tbl.at[pl.ds(r0, _ROWS_PER_TILE)])
      pltpu.sync_copy(z8_hbm.at[pl.ds(r0, _ROWS_PER_TILE)],
                      dacc.at[pl.ds(r0, _ROWS_PER_TILE)])
      pltpu.sync_copy(ones_hbm, onesb)
    else:
      pltpu.sync_copy(b_hbm, bbuf)

      def chunk(t, carry):
        cb = r0 + t * _CHUNK
        pltpu.sync_copy(s1_hbm.at[pl.ds(cb, _CHUNK)], p0c)
        pltpu.sync_copy(s1_hbm.at[pl.ds(_NPAD + cb, _CHUNK)], p1c)
        pltpu.sync_copy(xr_hbm.at[pl.ds(cb, _CHUNK)], xrc)
        pltpu.sync_copy(degp_hbm.at[pl.ds(cb, _CHUNK)], d0c)
        pltpu.sync_copy(degp_hbm.at[pl.ds(_NPAD + cb, _CHUNK)], d1c)

        def row(i, carry2):
          dv = d0c[i] + d1c[i]
          rvv = 1.0 / jnp.maximum(dv, 1.0)
          rvb = jnp.broadcast_to(rvv[0], (16,))
          for q in range(_HID // 16):
            col = pl.ds(16 * q, 16)
            v = ((p0c[i, col] + p1c[i, col]) * rvb + bbuf[col] + xrc[i, col])
            hc[i, col] = jnp.maximum(v, 0.0)
          return carry2

        lax.fori_loop(0, _CHUNK, row, 0)
        pltpu.sync_copy(hc, tbl.at[pl.ds(cb, _CHUNK)])
        return carry

      lax.fori_loop(0, _ROWS_PER_TILE // _CHUNK, chunk, 0)
      pltpu.sync_copy(
          tbl.at[pl.ds(r0 + c * (_ROWS_PER_TILE // 2), _ROWS_PER_TILE // 2)],
          h_hbm.at[pl.ds(r0 + c * (_ROWS_PER_TILE // 2), _ROWS_PER_TILE // 2)])
    plsc.subcore_barrier()

    base_row = wid * _SUBS_PER_W

    def group(g, carry):
      pltpu.sync_copy(src_hbm.at[pl.ds((base_row + g * _GRP) * _SUB, _GRP * _SUB)], srcb)
      pltpu.sync_copy(dst_hbm.at[pl.ds(base_row + g * _GRP, _GRP)], dstb)
      descs = [
          pltpu.async_copy(tbl.at[srcb.at[pl.ds(h * _NBUF * _SUB, _NBUF * _SUB)]],
                           rows[h], sems[h])
          for h in range(2)
      ]
      for h in range(2):
        descs[h].wait()
        for u in range(_NBUF):
          pltpu.sync_copy(rows[h].at[pl.ds(u * _SUB, _SUB)],
                          acc.at[dstb.at[h * _NBUF + u]], add=True)
          if with_deg:
            pltpu.sync_copy(onesb, dacc.at[dstb.at[h * _NBUF + u]], add=True)
      return carry

    lax.fori_loop(0, _NITER, group, 0)
    plsc.subcore_barrier()

    pltpu.sync_copy(acc.at[pl.ds(r0, _ROWS_PER_TILE)],
                    out_hbm.at[pl.ds(c * _NPAD + r0, _ROWS_PER_TILE)])
    if with_deg:
      pltpu.sync_copy(dacc.at[pl.ds(r0, _ROWS_PER_TILE)],
                      deg_hbm.at[pl.ds(c * _NPAD + r0, _ROWS_PER_TILE)])

  return pl.kernel(body, out_type=tuple(out_type),
                   mesh=mesh, scratch_types=scratch,
                   compiler_params=pltpu.CompilerParams(use_tc_tiling_on_sc=False))


_seg_deg = _make_seg_kernel(with_deg=True)
_seg = _make_seg_kernel(with_deg=False)


def _mm1_body(x_ref, w_ref, y_ref, yr_ref):
  xw = jnp.dot(x_ref[...], w_ref[...], preferred_element_type=jnp.float32)
  y_ref[...] = xw[:, :_HID]
  yr_ref[...] = xw[:, _HID:]


_mm1 = pl.pallas_call(
    _mm1_body,
    out_shape=(jax.ShapeDtypeStruct((_NPAD, _HID), jnp.float32),
               jax.ShapeDtypeStruct((_NPAD, _HID), jnp.float32)),
)


def _out_body(s2_ref, degp_ref, h_ref, wl_ref, wr_ref, b_ref, o_ref):
  s2 = s2_ref[: _NPAD] + s2_ref[_NPAD:]
  deg = degp_ref[: _NPAD] + degp_ref[_NPAD:]
  mean = s2 * (1.0 / jnp.maximum(deg[:, :1], 1.0))
  o_ref[...] = (
      jnp.dot(mean, wl_ref[...], preferred_element_type=jnp.float32)
      + jnp.dot(h_ref[...], wr_ref[...], preferred_element_type=jnp.float32)
      + b_ref[...]
  )


_outk = pl.pallas_call(
    _out_body,
    out_shape=jax.ShapeDtypeStruct((_NPAD, _D_OUT), jnp.float32),
)


@jax.jit
def kernel(x, edge_index, W_l1, b_l1, W_r1, W_l2, b_l2, W_r2):
  src = edge_index[0]
  dst = edge_index[1]
  pad_e = _EPAD - _E
  src_p = jnp.concatenate([src, jnp.full((pad_e,), _N, jnp.int32)])
  dst_pad = _N + (jnp.arange(pad_e, dtype=jnp.int32) % (_NPAD - _N))
  dst_p = jnp.concatenate([dst, dst_pad]).reshape(-1, _SUB)
  x_p = jnp.pad(x, ((0, _NPAD - _N), (0, 0)))

  z64 = jnp.zeros((_NPAD, _HID), jnp.float32)
  z8 = jnp.zeros((_NPAD, 16), jnp.float32)
  ones8 = jnp.ones((_SUB, 16), jnp.float32)

  w1 = jnp.concatenate([W_l1, W_r1], axis=1)
  y1, xr1 = _mm1(x_p, w1)

  s1_parts, deg_parts = _seg_deg(y1, src_p, dst_p, z64, z8, ones8)
  s2_parts, h = _seg(s1_parts, deg_parts, xr1, b_l1, src_p, dst_p, z64)
  out = _outk(s2_parts, deg_parts, h, W_l2, W_r2, b_l2.reshape(1, _D_OUT))
  return out[:_N]


# speedup vs baseline: 1.1793x; 1.1793x over previous
import functools

import jax
import jax.numpy as jnp
from jax import lax
from jax.experimental import pallas as pl
from jax.experimental.pallas import tpu as pltpu
from jax.experimental.pallas import tpu_sc as plsc

_N = 10000
_E = 320000
_D_IN = 128
_HID = 64
_D_OUT = 128

_NC = 2
_NS = 16
_NW = _NC * _NS
_SUB = 128
_SUBS_PER_W = 80
_EPAD = _NW * _SUBS_PER_W * _SUB
_NBUF = 2
_GRP = 2 * _NBUF + 1
_NITER = _SUBS_PER_W // _GRP
_NPAD = 10240
_ROWS_PER_TILE = _NPAD // _NS


def _make_seg_kernel(with_deg: bool):
  mesh = plsc.VectorSubcoreMesh(core_axis_name="c", subcore_axis_name="s")

  out_type = [jax.ShapeDtypeStruct((_NC * _NPAD, _HID), jnp.float32)]
  scratch = [
      pltpu.VMEM_SHARED((_NPAD, _HID), jnp.float32),
      pltpu.VMEM_SHARED((_NPAD, _HID), jnp.float32),
      pltpu.VMEM((_GRP * _SUB,), jnp.int32),
      pltpu.VMEM((_GRP, _SUB), jnp.int32),
      [pltpu.VMEM((_NBUF * _SUB, _HID), jnp.float32) for _ in range(2)],
      pltpu.VMEM((_SUB, _HID), jnp.float32),
      [pltpu.SemaphoreType.DMA, pltpu.SemaphoreType.DMA, pltpu.SemaphoreType.DMA],
  ]
  if with_deg:
    out_type.append(jax.ShapeDtypeStruct((_NC * _NPAD, 8), jnp.float32))
    scratch += [
        pltpu.VMEM_SHARED((_NPAD, 8), jnp.float32),
        pltpu.VMEM((_SUB, 8), jnp.float32),
    ]

  def body(*refs):
    if with_deg:
      (y_hbm, src_hbm, dst_hbm, z64_hbm, z8_hbm, ones_hbm,
       out_hbm, deg_hbm, acc, tbl, srcb, dstb, rows, rowsh, sems,
       dacc, onesb) = refs
    else:
      (y_hbm, src_hbm, dst_hbm, z64_hbm,
       out_hbm, acc, tbl, srcb, dstb, rows, rowsh, sems) = refs
    c = lax.axis_index("c")
    s = lax.axis_index("s")
    wid = s * _NC + c
    r0 = s * _ROWS_PER_TILE

    pltpu.sync_copy(z64_hbm.at[pl.ds(r0, _ROWS_PER_TILE)],
                    acc.at[pl.ds(r0, _ROWS_PER_TILE)])
    pltpu.sync_copy(y_hbm.at[pl.ds(r0, _ROWS_PER_TILE)],
                    tbl.at[pl.ds(r0, _ROWS_PER_TILE)])
    if with_deg:
      pltpu.sync_copy(z8_hbm.at[pl.ds(r0, _ROWS_PER_TILE)],
                      dacc.at[pl.ds(r0, _ROWS_PER_TILE)])
      pltpu.sync_copy(ones_hbm, onesb)
    plsc.subcore_barrier()

    base_row = wid * _SUBS_PER_W

    def group(g, carry):
      pltpu.sync_copy(src_hbm.at[pl.ds((base_row + g * _GRP) * _SUB, _GRP * _SUB)], srcb)
      pltpu.sync_copy(dst_hbm.at[pl.ds(base_row + g * _GRP, _GRP)], dstb)
      descs = [
          pltpu.async_copy(tbl.at[srcb.at[pl.ds(h * _NBUF * _SUB, _NBUF * _SUB)]],
                           rows[h], sems[h])
          for h in range(2)
      ]
      desch = pltpu.async_copy(
          y_hbm.at[srcb.at[pl.ds(2 * _NBUF * _SUB, _SUB)]], rowsh, sems[2])
      for h in range(2):
        descs[h].wait()
        for u in range(_NBUF):
          pltpu.sync_copy(rows[h].at[pl.ds(u * _SUB, _SUB)],
                          acc.at[dstb.at[h * _NBUF + u]], add=True)
          if with_deg:
            pltpu.sync_copy(onesb, dacc.at[dstb.at[h * _NBUF + u]], add=True)
      desch.wait()
      pltpu.sync_copy(rowsh, acc.at[dstb.at[2 * _NBUF]], add=True)
      if with_deg:
        pltpu.sync_copy(onesb, dacc.at[dstb.at[2 * _NBUF]], add=True)
      return carry

    lax.fori_loop(0, _NITER, group, 0)
    plsc.subcore_barrier()

    pltpu.sync_copy(acc.at[pl.ds(r0, _ROWS_PER_TILE)],
                    out_hbm.at[pl.ds(c * _NPAD + r0, _ROWS_PER_TILE)])
    if with_deg:
      pltpu.sync_copy(dacc.at[pl.ds(r0, _ROWS_PER_TILE)],
                      deg_hbm.at[pl.ds(c * _NPAD + r0, _ROWS_PER_TILE)])

  return pl.kernel(body, out_type=tuple(out_type) if with_deg else out_type[0],
                   mesh=mesh, scratch_types=scratch,
                   compiler_params=pltpu.CompilerParams(use_tc_tiling_on_sc=False))


_seg_deg = _make_seg_kernel(with_deg=True)
_seg = _make_seg_kernel(with_deg=False)


def _mm1_body(x_ref, w_ref, y_ref, yr_ref):
  xw = jnp.dot(x_ref[...], w_ref[...], preferred_element_type=jnp.float32)
  y_ref[...] = xw[:, :_HID]
  yr_ref[...] = xw[:, _HID:]


_mm1 = pl.pallas_call(
    _mm1_body,
    out_shape=(jax.ShapeDtypeStruct((_NPAD, _HID), jnp.float32),
               jax.ShapeDtypeStruct((_NPAD, _HID), jnp.float32)),
)


def _h_body(s1_ref, deg_ref, xr_ref, b_ref, h_ref, rdeg_ref):
  deg = deg_ref[: _NPAD] + deg_ref[_NPAD:]
  rdeg = 1.0 / jnp.maximum(deg, 1.0)
  s1 = s1_ref[: _NPAD] + s1_ref[_NPAD:]
  mean = s1 * rdeg[:, :1]
  h_ref[...] = jnp.maximum(mean + b_ref[...] + xr_ref[...], 0.0)
  rdeg_ref[...] = rdeg


_hk = pl.pallas_call(
    _h_body,
    out_shape=(jax.ShapeDtypeStruct((_NPAD, _HID), jnp.float32),
               jax.ShapeDtypeStruct((_NPAD, 8), jnp.float32)),
)


def _out_body(s2_ref, rdeg_ref, h_ref, wl_ref, wr_ref, b_ref, o_ref):
  s2 = s2_ref[: _NPAD] + s2_ref[_NPAD:]
  mean = s2 * rdeg_ref[:, :1]
  o_ref[...] = (
      jnp.dot(mean, wl_ref[...], preferred_element_type=jnp.float32)
      + jnp.dot(h_ref[...], wr_ref[...], preferred_element_type=jnp.float32)
      + b_ref[...]
  )


_outk = pl.pallas_call(
    _out_body,
    out_shape=jax.ShapeDtypeStruct((_NPAD, _D_OUT), jnp.float32),
)


@jax.jit
def kernel(x, edge_index, W_l1, b_l1, W_r1, W_l2, b_l2, W_r2):
  src = edge_index[0]
  dst = edge_index[1]
  pad_e = _EPAD - _E
  src_p = jnp.concatenate([src, jnp.full((pad_e,), _N, jnp.int32)])
  dst_pad = _N + (jnp.arange(pad_e, dtype=jnp.int32) % (_NPAD - _N))
  dst_p = jnp.concatenate([dst, dst_pad]).reshape(-1, _SUB)
  x_p = jnp.pad(x, ((0, _NPAD - _N), (0, 0)))

  z64 = jnp.zeros((_NPAD, _HID), jnp.float32)
  z8 = jnp.zeros((_NPAD, 8), jnp.float32)
  ones8 = jnp.ones((_SUB, 8), jnp.float32)

  w1 = jnp.concatenate([W_l1, W_r1], axis=1)
  y1, xr1 = _mm1(x_p, w1)

  s1_parts, deg_parts = _seg_deg(y1, src_p, dst_p, z64, z8, ones8)
  h, rdeg = _hk(s1_parts, deg_parts, xr1, b_l1.reshape(1, _HID))

  s2_parts = _seg(h, src_p, dst_p, z64)
  out = _outk(s2_parts, rdeg, h, W_l2, W_r2, b_l2.reshape(1, _D_OUT))
  return out[:_N]


# speedup vs baseline: 1.1897x; 1.0088x over previous
import functools

import jax
import jax.numpy as jnp
from jax import lax
from jax.experimental import pallas as pl
from jax.experimental.pallas import tpu as pltpu
from jax.experimental.pallas import tpu_sc as plsc

_N = 10000
_E = 320000
_D_IN = 128
_HID = 64
_D_OUT = 128

_NC = 2
_NS = 16
_NW = _NC * _NS
_SUB = 128
_SUBS_PER_W = 80
_EPAD = _NW * _SUBS_PER_W * _SUB
_NBUF = 2
_GRP = 2 * _NBUF
_NITER = _SUBS_PER_W // _GRP
_NPAD = 10240
_ROWS_PER_TILE = _NPAD // _NS


def _make_seg_kernel(with_deg: bool):
  mesh = plsc.VectorSubcoreMesh(core_axis_name="c", subcore_axis_name="s")

  out_type = [jax.ShapeDtypeStruct((_NC * _NPAD, _HID), jnp.float32)]
  scratch = [
      pltpu.VMEM_SHARED((_NPAD, _HID), jnp.float32),
      pltpu.VMEM_SHARED((_NPAD, _HID), jnp.float32),
      pltpu.VMEM((_GRP * _SUB,), jnp.int32),
      pltpu.VMEM((_GRP, _SUB), jnp.int32),
      [pltpu.VMEM((_NBUF * _SUB, _HID), jnp.float32) for _ in range(2)],
      [pltpu.SemaphoreType.DMA, pltpu.SemaphoreType.DMA],
  ]
  if with_deg:
    out_type.append(jax.ShapeDtypeStruct((_NC * _NPAD, 8), jnp.float32))
    scratch += [
        pltpu.VMEM_SHARED((_NPAD, 8), jnp.float32),
        pltpu.VMEM((_SUB, 8), jnp.float32),
    ]

  def body(*refs):
    if with_deg:
      (y_hbm, src_hbm, dst_hbm, z64_hbm, z8_hbm, ones_hbm,
       out_hbm, deg_hbm, acc, tbl, srcb, dstb, rows, sems, dacc, onesb) = refs
    else:
      (y_hbm, src_hbm, dst_hbm, z64_hbm,
       out_hbm, acc, tbl, srcb, dstb, rows, sems) = refs
    c = lax.axis_index("c")
    s = lax.axis_index("s")
    wid = s * _NC + c
    r0 = s * _ROWS_PER_TILE

    pltpu.sync_copy(z64_hbm.at[pl.ds(r0, _ROWS_PER_TILE)],
                    acc.at[pl.ds(r0, _ROWS_PER_TILE)])
    pltpu.sync_copy(y_hbm.at[pl.ds(r0, _ROWS_PER_TILE)],
                    tbl.at[pl.ds(r0, _ROWS_PER_TILE)])
    if with_deg:
      pltpu.sync_copy(z8_hbm.at[pl.ds(r0, _ROWS_PER_TILE)],
                      dacc.at[pl.ds(r0, _ROWS_PER_TILE)])
      pltpu.sync_copy(ones_hbm, onesb)
    plsc.subcore_barrier()

    base_row = wid * _SUBS_PER_W

    def group(g, carry):
      pltpu.sync_copy(src_hbm.at[pl.ds((base_row + g * _GRP) * _SUB, _GRP * _SUB)], srcb)
      pltpu.sync_copy(dst_hbm.at[pl.ds(base_row + g * _GRP, _GRP)], dstb)
      descs = [
          pltpu.async_copy(tbl.at[srcb.at[pl.ds(h * _NBUF * _SUB, _NBUF * _SUB)]],
                           rows[h], sems[h])
          for h in range(2)
      ]
      for h in range(2):
        descs[h].wait()
        for u in range(_NBUF):
          pltpu.sync_copy(rows[h].at[pl.ds(u * _SUB, _SUB)],
                          acc.at[dstb.at[h * _NBUF + u]], add=True)
          if with_deg:
            pltpu.sync_copy(onesb, dacc.at[dstb.at[h * _NBUF + u]], add=True)
      return carry

    lax.fori_loop(0, _NITER, group, 0)
    plsc.subcore_barrier()

    pltpu.sync_copy(acc.at[pl.ds(r0, _ROWS_PER_TILE)],
                    out_hbm.at[pl.ds(c * _NPAD + r0, _ROWS_PER_TILE)])
    if with_deg:
      pltpu.sync_copy(dacc.at[pl.ds(r0, _ROWS_PER_TILE)],
                      deg_hbm.at[pl.ds(c * _NPAD + r0, _ROWS_PER_TILE)])

  return pl.kernel(body, out_type=tuple(out_type) if with_deg else out_type[0],
                   mesh=mesh, scratch_types=scratch,
                   compiler_params=pltpu.CompilerParams(use_tc_tiling_on_sc=False))


_seg_deg = _make_seg_kernel(with_deg=True)
_seg = _make_seg_kernel(with_deg=False)


def _mm1_body(x_ref, w_ref, y_ref, yr_ref):
  xw = jnp.dot(x_ref[...], w_ref[...], preferred_element_type=jnp.float32)
  y_ref[...] = xw[:, :_HID]
  yr_ref[...] = xw[:, _HID:]


_mm1 = pl.pallas_call(
    _mm1_body,
    out_shape=(jax.ShapeDtypeStruct((_NPAD, _HID), jnp.float32),
               jax.ShapeDtypeStruct((_NPAD, _HID), jnp.float32)),
)


def _h_body(s1_ref, deg_ref, xr_ref, b_ref, h_ref, rdeg_ref):
  deg = deg_ref[: _NPAD] + deg_ref[_NPAD:]
  rdeg = 1.0 / jnp.maximum(deg, 1.0)
  s1 = s1_ref[: _NPAD] + s1_ref[_NPAD:]
  mean = s1 * rdeg[:, :1]
  h_ref[...] = jnp.maximum(mean + b_ref[...] + xr_ref[...], 0.0)
  rdeg_ref[...] = rdeg


_hk = pl.pallas_call(
    _h_body,
    out_shape=(jax.ShapeDtypeStruct((_NPAD, _HID), jnp.float32),
               jax.ShapeDtypeStruct((_NPAD, 8), jnp.float32)),
)


def _out_body(s2_ref, rdeg_ref, h_ref, wl_ref, wr_ref, b_ref, o_ref):
  s2 = s2_ref[: _NPAD] + s2_ref[_NPAD:]
  mean = s2 * rdeg_ref[:, :1]
  o_ref[...] = (
      jnp.dot(mean, wl_ref[...], preferred_element_type=jnp.float32)
      + jnp.dot(h_ref[...], wr_ref[...], preferred_element_type=jnp.float32)
      + b_ref[...]
  )


_outk = pl.pallas_call(
    _out_body,
    out_shape=jax.ShapeDtypeStruct((_NPAD, _D_OUT), jnp.float32),
)


@jax.jit
def kernel(x, edge_index, W_l1, b_l1, W_r1, W_l2, b_l2, W_r2):
  src = edge_index[0]
  dst = edge_index[1]
  pad_e = _EPAD - _E
  src_p = jnp.concatenate([src, jnp.full((pad_e,), _N, jnp.int32)])
  dst_pad = _N + (jnp.arange(pad_e, dtype=jnp.int32) % (_NPAD - _N))
  dst_p = jnp.concatenate([dst, dst_pad]).reshape(-1, _SUB)
  x_p = jnp.pad(x, ((0, _NPAD - _N), (0, 0)))

  z64 = jnp.zeros((_NPAD, _HID), jnp.float32)
  z8 = jnp.zeros((_NPAD, 8), jnp.float32)
  ones8 = jnp.ones((_SUB, 8), jnp.float32)

  w1 = jnp.concatenate([W_l1, W_r1], axis=1)
  y1, xr1 = _mm1(x_p, w1)

  s1_parts, deg_parts = _seg_deg(y1, src_p, dst_p, z64, z8, ones8)
  h, rdeg = _hk(s1_parts, deg_parts, xr1, b_l1.reshape(1, _HID))

  s2_parts = _seg(h, src_p, dst_p, z64)
  out = _outk(s2_parts, rdeg, h, W_l2, W_r2, b_l2.reshape(1, _D_OUT))
  return out[:_N]
